# baseline (speedup 1.0000x reference)
import jax
import jax.numpy as jnp
from jax.experimental import pallas as pl
from jax.experimental.pallas import tpu as pltpu

_NEG_BIG = -1e30


def _fused_head_kernel(x_ref, w_ref, b_ref, lbl_ref,
                       loss_ref, hit1_ref, hit5_ref):
    x = x_ref[...]
    hw = x.shape[2]
    pooled = jnp.sum(x, axis=2) * (1.0 / hw)

    logits = jnp.dot(pooled, w_ref[...],
                     preferred_element_type=jnp.float32) + b_ref[...]

    m = jnp.max(logits, axis=1, keepdims=True)
    lse = m + jnp.log(jnp.sum(jnp.exp(logits - m), axis=1, keepdims=True))
    tn, kp = logits.shape
    cls_iota = jax.lax.broadcasted_iota(jnp.int32, (tn, kp), 1)
    lbl = lbl_ref[...]
    picked = jnp.sum(jnp.where(cls_iota == lbl, logits, 0.0),
                     axis=1, keepdims=True)
    loss_ref[...] = lse - picked

    n_greater = jnp.sum((logits > picked).astype(jnp.float32),
                        axis=1, keepdims=True)
    n_eq_before = jnp.sum(((logits == picked) & (cls_iota < lbl))
                          .astype(jnp.float32), axis=1, keepdims=True)
    rank = n_greater + n_eq_before
    hit1_ref[...] = (rank < 1.0).astype(jnp.float32)
    hit5_ref[...] = (rank < 5.0).astype(jnp.float32)


def kernel(x, w, b, gt_label):
    N, C, H, W = x.shape
    K = w.shape[1]
    HW = H * W

    x3 = x.reshape(N, C, HW)

    K_pad = max(128, ((K + 127) // 128) * 128)
    w_pad = jnp.pad(w, ((0, 0), (0, K_pad - K)))
    b_pad = jnp.pad(b.reshape(1, K), ((0, 0), (0, K_pad - K)),
                    constant_values=_NEG_BIG)
    lbl2 = gt_label.astype(jnp.int32).reshape(N, 1)

    TILE_N = min(N, 16)
    grid = (pl.cdiv(N, TILE_N),)

    loss, hit1, hit5 = pl.pallas_call(
        _fused_head_kernel,
        out_shape=(
            jax.ShapeDtypeStruct((N, 1), jnp.float32),
            jax.ShapeDtypeStruct((N, 1), jnp.float32),
            jax.ShapeDtypeStruct((N, 1), jnp.float32),
        ),
        grid=grid,
        in_specs=[
            pl.BlockSpec((TILE_N, C, HW), lambda i: (i, 0, 0)),
            pl.BlockSpec((C, K_pad), lambda i: (0, 0)),
            pl.BlockSpec((1, K_pad), lambda i: (0, 0)),
            pl.BlockSpec((TILE_N, 1), lambda i: (i, 0)),
        ],
        out_specs=(
            pl.BlockSpec((TILE_N, 1), lambda i: (i, 0)),
            pl.BlockSpec((TILE_N, 1), lambda i: (i, 0)),
            pl.BlockSpec((TILE_N, 1), lambda i: (i, 0)),
        ),
        compiler_params=pltpu.CompilerParams(
            dimension_semantics=("parallel",),
            vmem_limit_bytes=48 * 1024 * 1024,
        ),
    )(x3, w_pad, b_pad, lbl2)

    return {
        "loss": jnp.mean(loss),
        "accuracy": {
            "top-1": jnp.mean(hit1) * 100.0,
            "top-5": jnp.mean(hit5) * 100.0,
        },
    }
